# baseline (speedup 1.0000x reference)
import jax
import jax.numpy as jnp
from jax.experimental import pallas as pl

NUM_CLASSES = 1000
ROWS = 4096 * 20
BLOCK_ROWS = 512


def _onehot_body(cls_ref, out_ref):
    cls = cls_ref[...]
    iota = jax.lax.broadcasted_iota(cls.dtype, (BLOCK_ROWS, NUM_CLASSES), 1)
    out_ref[...] = (cls == iota).astype(jnp.int32)


def kernel(classes):
    flat = classes.reshape(ROWS, 1)
    out = pl.pallas_call(
        _onehot_body,
        grid=(ROWS // BLOCK_ROWS,),
        in_specs=[pl.BlockSpec((BLOCK_ROWS, 1), lambda i: (i, 0))],
        out_specs=pl.BlockSpec((BLOCK_ROWS, NUM_CLASSES), lambda i: (i, 0)),
        out_shape=jax.ShapeDtypeStruct((ROWS, NUM_CLASSES), jnp.int32),
    )(flat)
    return out.reshape(classes.shape + (NUM_CLASSES,))


# speedup vs baseline: 1.6983x; 1.6983x over previous
import jax
import jax.numpy as jnp
from jax.experimental import pallas as pl

NUM_CLASSES = 1000
B0, B1 = 4096, 20
BLOCK = 32


def _onehot_body(cls_ref, out_ref):
    cls = cls_ref[...][..., None]
    iota = jax.lax.broadcasted_iota(cls.dtype, (BLOCK, B1, NUM_CLASSES), 2)
    out_ref[...] = (cls == iota).astype(jnp.int32)


def kernel(classes):
    return pl.pallas_call(
        _onehot_body,
        grid=(B0 // BLOCK,),
        in_specs=[pl.BlockSpec((BLOCK, B1), lambda i: (i, 0))],
        out_specs=pl.BlockSpec((BLOCK, B1, NUM_CLASSES), lambda i: (i, 0, 0)),
        out_shape=jax.ShapeDtypeStruct((B0, B1, NUM_CLASSES), jnp.int32),
    )(classes)


# speedup vs baseline: 1.7341x; 1.0211x over previous
import jax
import jax.numpy as jnp
from jax.experimental import pallas as pl

NUM_CLASSES = 1000
B0, B1 = 4096, 20
BLOCK = 128


def _onehot_body(cls_ref, out_ref):
    cls = cls_ref[...][..., None]
    iota = jax.lax.broadcasted_iota(cls.dtype, (BLOCK, B1, NUM_CLASSES), 2)
    out_ref[...] = (cls == iota).astype(jnp.int32)


def kernel(classes):
    return pl.pallas_call(
        _onehot_body,
        grid=(B0 // BLOCK,),
        in_specs=[pl.BlockSpec((BLOCK, B1), lambda i: (i, 0))],
        out_specs=pl.BlockSpec((BLOCK, B1, NUM_CLASSES), lambda i: (i, 0, 0)),
        out_shape=jax.ShapeDtypeStruct((B0, B1, NUM_CLASSES), jnp.int32),
    )(classes)
